# Initial kernel scaffold; baseline (speedup 1.0000x reference)
import jax
import jax.numpy as jnp
from jax.experimental import pallas as pl


def kernel(tokens, prompt_table, wte_table):
    raise NotImplementedError("write your pallas kernel here")


# speedup vs baseline: 4.4911x; 4.4911x over previous
import functools

import jax
import jax.numpy as jnp
from jax import lax
from jax.experimental import pallas as pl
from jax.experimental.pallas import tpu as pltpu
from jax.experimental.pallas import tpu_sc as plsc

V = 100000
H = 64
B = 4096
L = 200
P = 10
BIAS = 1

_info = plsc.get_sparse_core_info()
_NC, _NS = _info.num_cores, _info.num_subcores
_NW = _NC * _NS
_PER_W = B // _NW


def _body(tokens_hbm, prompt_hbm, wte_hbm, out_hbm,
          tok_v, widx_v, pidx_v, pfix, staging, sem):
    wid = lax.axis_index("s") * _NC + lax.axis_index("c")
    o = lax.iota(jnp.int32, 16)
    s_w = jnp.where(o <= P, 0, o)
    s_p = jnp.minimum(o + 1, P)

    def step(i, carry):
        base = (wid * _PER_W + i) * L
        pltpu.sync_copy(tokens_hbm.at[pl.ds(base, L)], tok_v)
        t0 = tok_v[pl.ds(0, 16)]
        widx_v[...] = t0.at[s_w].get(mode="promise_in_bounds")
        pidx_v[...] = t0.at[s_p].get(mode="promise_in_bounds") - V
        c1 = pltpu.async_copy(wte_hbm.at[widx_v], staging.at[pl.ds(0, 16)], sem)
        c2 = pltpu.async_copy(prompt_hbm.at[pidx_v], pfix, sem)
        c3 = pltpu.async_copy(wte_hbm.at[tok_v.at[pl.ds(16, 96)]],
                              staging.at[pl.ds(16, 96)], sem)
        c4 = pltpu.async_copy(wte_hbm.at[tok_v.at[pl.ds(112, 88)]],
                              staging.at[pl.ds(112, 88)], sem)
        c1.wait(); c2.wait(); c3.wait(); c4.wait()
        for r in range(P):
            for c in range(H // 16):
                staging[r, pl.ds(c * 16, 16)] = pfix[r, pl.ds(c * 16, 16)]
        pltpu.sync_copy(staging, out_hbm.at[pl.ds(base, L)])
        return carry

    lax.fori_loop(0, _PER_W, step, 0)


@jax.jit
def kernel(tokens, prompt_table, wte_table):
    mesh = plsc.VectorSubcoreMesh(core_axis_name="c", subcore_axis_name="s")
    run = functools.partial(
        pl.kernel,
        mesh=mesh,
        compiler_params=pltpu.CompilerParams(use_tc_tiling_on_sc=False),
        out_type=jax.ShapeDtypeStruct((B * L, H), jnp.float32),
        scratch_types=[
            pltpu.VMEM((L,), jnp.int32),
            pltpu.VMEM((16,), jnp.int32),
            pltpu.VMEM((16,), jnp.int32),
            pltpu.VMEM((16, H), jnp.float32),
            pltpu.VMEM((L, H), jnp.float32),
            pltpu.SemaphoreType.DMA,
        ],
    )(_body)
    out = run(tokens.reshape(B * L), prompt_table, wte_table)
    return out.reshape(B, L, H)
